# baseline (device time: 10466 ns/iter reference)
import jax
import jax.numpy as jnp
from jax import lax
from jax.experimental import pallas as pl
from jax.experimental.pallas import tpu as pltpu

N_DEV = 4
N_EXP = 8
E_LOCAL = 2
N_TOK = 256
D_IN = 128
D_OUT = 256
ROWS = N_TOK // N_DEV


def kernel(x, router_W, route_idx, expert_W, shared_W):
    def body(x_ref, rw_ref, idx_ref, ew_ref, sw_ref, out_ref,
             partial_ref, recv_ref, send_sems, recv_sems):
        my = lax.axis_index("i")

        xv = x_ref[:, :]
        scores = jnp.dot(xv, rw_ref[:, :], preferred_element_type=jnp.float32)
        m = jnp.max(scores, axis=1, keepdims=True)
        p = jnp.exp(scores - m)
        probs = p / jnp.sum(p, axis=1, keepdims=True)
        eids = lax.broadcasted_iota(jnp.int32, (N_TOK, N_EXP), 1)
        gated = jnp.where(eids == idx_ref[:, :], probs, 0.0)

        partial = jnp.zeros((N_TOK, D_OUT), jnp.float32)
        for l in range(E_LOCAL):
            g = my * E_LOCAL + l
            gate = jnp.sum(jnp.where(eids == g, gated, 0.0), axis=1,
                           keepdims=True)
            partial += jnp.dot(xv * gate, ew_ref[l, :, :],
                               preferred_element_type=jnp.float32)
        partial_ref[...] = partial.reshape(N_DEV, ROWS, D_OUT)

        bsem = pltpu.get_barrier_semaphore()
        for d in range(1, N_DEV):
            pl.semaphore_signal(
                bsem, inc=1,
                device_id=(lax.rem(my + d, N_DEV),),
                device_id_type=pl.DeviceIdType.MESH,
            )
        pl.semaphore_wait(bsem, N_DEV - 1)

        rdmas = []
        for d in range(1, N_DEV):
            tgt = lax.rem(my + d, N_DEV)
            rdma = pltpu.make_async_remote_copy(
                src_ref=partial_ref.at[tgt],
                dst_ref=recv_ref.at[d - 1],
                send_sem=send_sems.at[d - 1],
                recv_sem=recv_sems.at[d - 1],
                device_id=(tgt,),
                device_id_type=pl.DeviceIdType.MESH,
            )
            rdma.start()
            rdmas.append(rdma)

        x_my = x_ref[pl.ds(my * ROWS, ROWS), :]
        acc = jnp.dot(x_my, sw_ref[:, :], preferred_element_type=jnp.float32)
        acc = acc + partial_ref[my]

        for r in rdmas:
            r.wait_recv()
        acc = acc + recv_ref[0] + recv_ref[1] + recv_ref[2]
        out_ref[...] = acc

        for r in rdmas:
            r.wait_send()

    return pl.pallas_call(
        body,
        out_shape=jax.ShapeDtypeStruct((ROWS, D_OUT), jnp.float32),
        in_specs=[pl.BlockSpec(memory_space=pltpu.VMEM)] * 5,
        out_specs=pl.BlockSpec(memory_space=pltpu.VMEM),
        scratch_shapes=[
            pltpu.VMEM((N_DEV, ROWS, D_OUT), jnp.float32),
            pltpu.VMEM((N_DEV - 1, ROWS, D_OUT), jnp.float32),
            pltpu.SemaphoreType.DMA((N_DEV - 1,)),
            pltpu.SemaphoreType.DMA((N_DEV - 1,)),
        ],
        compiler_params=pltpu.CompilerParams(collective_id=0),
    )(x, router_W, route_idx, expert_W, shared_W)
